# Initial kernel scaffold; baseline (speedup 1.0000x reference)
import jax
import jax.numpy as jnp
from jax.experimental import pallas as pl


def kernel(x, edge_index, W1, W2):
    raise NotImplementedError("write your pallas kernel here")


# speedup vs baseline: 5.8630x; 5.8630x over previous
import functools

import jax
import jax.numpy as jnp
from jax import lax
from jax.experimental import pallas as pl
from jax.experimental.pallas import tpu as pltpu
from jax.experimental.pallas import tpu_sc as plsc

N = 50000
E = 800000
IN_F = 69
F1 = 96
CF1 = 24
HID = 128
OUT = 64
CF2 = 16

NC = 2
NT = 16

EPR = 80
E_PAD = 819200
ROWS = E_PAD // EPR
RPT = ROWS // NT
IDXR = 128
KG = 8
N_OUTER = RPT // IDXR
N_INNER = IDXR // KG

NAGG = 51200
STRIPE = NAGG // NT
ZR = 128
NZ = STRIPE // ZR

RB = 400
GRID = N // RB


def _deg_pass(srcd2d, dst2d):
    mesh = plsc.VectorSubcoreMesh(core_axis_name="c", subcore_axis_name="s")

    @functools.partial(
        pl.kernel,
        out_type=jax.ShapeDtypeStruct((NC, NAGG), jnp.float32),
        mesh=mesh,
        compiler_params=pltpu.CompilerParams(use_tc_tiling_on_sc=False),
        scratch_types=[
            pltpu.VMEM((IDXR, EPR), jnp.int32),
            pltpu.VMEM((EPR,), jnp.float32),
            pltpu.VMEM((STRIPE,), jnp.float32),
            pltpu.VMEM_SHARED((NAGG,), jnp.float32),
            pltpu.SemaphoreType.DMA,
        ],
    )
    def k(src_h, dst_h, out_h, idxv, ones_v, zflat, deg_sh, sem):
        c = lax.axis_index("c")
        s = lax.axis_index("s")

        zero16 = jnp.zeros((16,), jnp.float32)
        one16 = jnp.ones((16,), jnp.float32)

        def zfill(i, _):
            zflat[pl.ds(i * 16, 16)] = zero16
            return 0

        lax.fori_loop(0, STRIPE // 16, zfill, 0)
        for b in range(EPR // 16):
            ones_v[pl.ds(b * 16, 16)] = one16
        pltpu.sync_copy(zflat, deg_sh.at[pl.ds(s * STRIPE, STRIPE)])
        plsc.subcore_barrier()

        def process(idx_h):
            def outer(o, _):
                r0 = s * RPT + o * IDXR
                pltpu.sync_copy(idx_h.at[pl.ds(r0, IDXR)], idxv)

                def inner(g, _):
                    descs = [
                        pltpu.async_copy(
                            ones_v, deg_sh.at[idxv.at[g * KG + b]], sem, add=True
                        )
                        for b in range(KG)
                    ]
                    for d in descs:
                        d.wait()
                    return 0

                lax.fori_loop(0, N_INNER, inner, 0)
                return 0

            lax.fori_loop(0, N_OUTER, outer, 0)

        @pl.when(c == 0)
        def _():
            process(src_h)

        @pl.when(c == 1)
        def _():
            process(dst_h)

        plsc.subcore_barrier()
        pltpu.sync_copy(
            deg_sh.at[pl.ds(s * STRIPE, STRIPE)], out_h.at[c, pl.ds(s * STRIPE, STRIPE)]
        )

    return k(srcd2d, dst2d)


def _edge_pass(tabs, src2d, dst2d, zrows, F):
    mesh = plsc.VectorSubcoreMesh(core_axis_name="c", subcore_axis_name="s")

    @functools.partial(
        pl.kernel,
        out_type=jax.ShapeDtypeStruct((4, NAGG, F), jnp.float32),
        mesh=mesh,
        compiler_params=pltpu.CompilerParams(use_tc_tiling_on_sc=False),
        scratch_types=[
            pltpu.VMEM((IDXR, EPR), jnp.int32),
            pltpu.VMEM((IDXR, EPR), jnp.int32),
            pltpu.VMEM((KG, EPR, F), jnp.float32),
            pltpu.VMEM((ZR, F), jnp.float32),
            pltpu.VMEM_SHARED((NAGG, F), jnp.float32),
            pltpu.SemaphoreType.DMA,
            pltpu.SemaphoreType.DMA,
        ],
    )
    def k(t0_h, t1_h, t2_h, t3_h, src_h, dst_h, z_h, out_h,
          sidx, didx, rows, zbuf, agg, gsem, ssem):
        c = lax.axis_index("c")
        s = lax.axis_index("s")

        pltpu.sync_copy(z_h, zbuf)

        def zero_stripe():
            def zloop(i, _):
                pltpu.sync_copy(zbuf, agg.at[pl.ds(s * STRIPE + i * ZR, ZR)])
                return 0

            lax.fori_loop(0, NZ, zloop, 0)

        def process(tab_h):
            def outer(o, _):
                r0 = s * RPT + o * IDXR
                pltpu.sync_copy(src_h.at[pl.ds(r0, IDXR)], sidx)
                pltpu.sync_copy(dst_h.at[pl.ds(r0, IDXR)], didx)

                def inner(g, _):
                    j0 = g * KG
                    gds = [
                        pltpu.async_copy(tab_h.at[sidx.at[j0 + b]], rows.at[b], gsem)
                        for b in range(KG)
                    ]
                    for d in gds:
                        d.wait()
                    sds = [
                        pltpu.async_copy(
                            rows.at[b], agg.at[didx.at[j0 + b]], ssem, add=True
                        )
                        for b in range(KG)
                    ]
                    for d in sds:
                        d.wait()
                    return 0

                lax.fori_loop(0, N_INNER, inner, 0)
                return 0

            lax.fori_loop(0, N_OUTER, outer, 0)

        def writeout(q):
            def wloop(i, _):
                off = s * STRIPE + i * ZR
                pltpu.sync_copy(agg.at[pl.ds(off, ZR)], out_h.at[q, pl.ds(off, ZR)])
                return 0

            lax.fori_loop(0, NZ, wloop, 0)

        zero_stripe()
        plsc.subcore_barrier()

        @pl.when(c == 0)
        def _():
            process(t0_h)

        @pl.when(c == 1)
        def _():
            process(t2_h)

        plsc.subcore_barrier()
        writeout(2 * c)
        zero_stripe()
        plsc.subcore_barrier()

        @pl.when(c == 0)
        def _():
            process(t1_h)

        @pl.when(c == 1)
        def _():
            process(t3_h)

        plsc.subcore_barrier()
        writeout(2 * c + 1)

    return k(tabs[0], tabs[1], tabs[2], tabs[3], src2d, dst2d, zrows)


def _k2_body(deg_ref, x0_ref, x1_ref, x2_ref, x3_ref,
             o0_ref, o1_ref, o2_ref, o3_ref, ns_ref, nd_ref):
    d = deg_ref[...]
    ns = lax.rsqrt(jnp.maximum(d[:, 0:1], 1.0))
    nd = lax.rsqrt(jnp.maximum(d[:, 1:2], 1.0))
    o0_ref[...] = x0_ref[...] * ns
    o1_ref[...] = x1_ref[...] * ns
    o2_ref[...] = x2_ref[...] * ns
    o3_ref[...] = x3_ref[...] * ns
    ns_ref[...] = ns
    nd_ref[...] = nd


def _k2(deg_t, xqs):
    xspec = pl.BlockSpec((RB, CF1), lambda i: (i, 0))
    nspec = pl.BlockSpec((RB, 1), lambda i: (i, 0))
    return pl.pallas_call(
        _k2_body,
        grid=(GRID,),
        in_specs=[pl.BlockSpec((RB, NC), lambda i: (i, 0))] + [xspec] * 4,
        out_specs=[xspec] * 4 + [nspec, nspec],
        out_shape=[jax.ShapeDtypeStruct((N, CF1), jnp.float32)] * 4
        + [jax.ShapeDtypeStruct((N, 1), jnp.float32)] * 2,
    )(deg_t, *xqs)


def _k4_body(a0_ref, a1_ref, a2_ref, a3_ref, ns_ref, nd_ref,
             w10_ref, w11_ref, w12_ref, w13_ref,
             w20_ref, w21_ref, w22_ref, w23_ref,
             p0_ref, p1_ref, p2_ref, p3_ref):
    nd = nd_ref[...]
    z = jnp.dot(a0_ref[0] * nd, w10_ref[...], preferred_element_type=jnp.float32)
    z = z + jnp.dot(a1_ref[0] * nd, w11_ref[...], preferred_element_type=jnp.float32)
    z = z + jnp.dot(a2_ref[0] * nd, w12_ref[...], preferred_element_type=jnp.float32)
    z = z + jnp.dot(a3_ref[0] * nd, w13_ref[...], preferred_element_type=jnp.float32)
    z = jnp.maximum(z, 0.0) * ns_ref[...]
    p0_ref[...] = jnp.dot(z, w20_ref[...], preferred_element_type=jnp.float32)
    p1_ref[...] = jnp.dot(z, w21_ref[...], preferred_element_type=jnp.float32)
    p2_ref[...] = jnp.dot(z, w22_ref[...], preferred_element_type=jnp.float32)
    p3_ref[...] = jnp.dot(z, w23_ref[...], preferred_element_type=jnp.float32)


def _k4(agg1, ns, nd, w1qs, w2qs):
    aspec = [
        pl.BlockSpec((1, RB, CF1), (lambda q: (lambda i: (q, i, 0)))(q))
        for q in range(4)
    ]
    nspec = pl.BlockSpec((RB, 1), lambda i: (i, 0))
    return pl.pallas_call(
        _k4_body,
        grid=(GRID,),
        in_specs=aspec
        + [nspec, nspec]
        + [pl.BlockSpec((CF1, HID), lambda i: (0, 0))] * 4
        + [pl.BlockSpec((HID, CF2), lambda i: (0, 0))] * 4,
        out_specs=[pl.BlockSpec((RB, CF2), lambda i: (i, 0))] * 4,
        out_shape=[jax.ShapeDtypeStruct((N, CF2), jnp.float32)] * 4,
    )(agg1, agg1, agg1, agg1, ns, nd, *w1qs, *w2qs)


def _k6_body(a0_ref, a1_ref, a2_ref, a3_ref, nd_ref, o_ref):
    i = pl.program_id(0)
    nd = nd_ref[...]
    ms = [
        jnp.max(a_ref[0] * nd, axis=0, keepdims=True)
        for a_ref in (a0_ref, a1_ref, a2_ref, a3_ref)
    ]
    m = jnp.concatenate(ms, axis=0)

    @pl.when(i == 0)
    def _():
        o_ref[...] = m

    @pl.when(i > 0)
    def _():
        o_ref[...] = jnp.maximum(o_ref[...], m)

    @pl.when(i == GRID - 1)
    def _():
        o_ref[...] = jnp.maximum(o_ref[...], 0.0)


def _k6(agg2, nd):
    aspec = [
        pl.BlockSpec((1, RB, CF2), (lambda q: (lambda i: (q, i, 0)))(q))
        for q in range(4)
    ]
    return pl.pallas_call(
        _k6_body,
        grid=(GRID,),
        in_specs=aspec + [pl.BlockSpec((RB, 1), lambda i: (i, 0))],
        out_specs=pl.BlockSpec((4, CF2), lambda i: (0, 0)),
        out_shape=jax.ShapeDtypeStruct((4, CF2), jnp.float32),
    )(agg2, agg2, agg2, agg2, nd)


def kernel(x, edge_index, W1, W2):
    ei = edge_index.astype(jnp.int32)
    src = ei[0]
    dst = ei[1]

    npad = E_PAD - E
    ar = jnp.arange(npad, dtype=jnp.int32)
    trash = N + (ar % (NAGG - N - 8))
    src_deg2d = jnp.concatenate([src, trash]).reshape(ROWS, EPR)
    src_edge2d = jnp.concatenate([src, ar % 1024]).reshape(ROWS, EPR)
    dst2d = jnp.concatenate([dst, trash]).reshape(ROWS, EPR)

    xp = jnp.pad(x, ((0, 0), (0, F1 - IN_F)))
    xqs = [xp[:, q * CF1:(q + 1) * CF1] for q in range(4)]
    w1p = jnp.pad(W1, ((0, F1 - IN_F), (0, 0)))
    w1qs = [w1p[q * CF1:(q + 1) * CF1] for q in range(4)]
    w2qs = [W2[:, q * CF2:(q + 1) * CF2] for q in range(4)]
    z24 = jnp.zeros((ZR, CF1), jnp.float32)
    z16 = jnp.zeros((ZR, CF2), jnp.float32)

    deg = _deg_pass(src_deg2d, dst2d)
    deg_t = deg[:, :N].T
    k2out = _k2(deg_t, xqs)
    xn_qs, ns, nd = k2out[:4], k2out[4], k2out[5]
    agg1 = _edge_pass(xn_qs, src_edge2d, dst2d, z24, CF1)[:, :N]
    pqs = _k4(agg1, ns, nd, w1qs, w2qs)
    agg2 = _edge_pass(pqs, src_edge2d, dst2d, z16, CF2)[:, :N]
    out = _k6(agg2, nd)
    return out.reshape(1, OUT)


# speedup vs baseline: 6.4013x; 1.0918x over previous
import functools

import jax
import jax.numpy as jnp
from jax import lax
from jax.experimental import pallas as pl
from jax.experimental.pallas import tpu as pltpu
from jax.experimental.pallas import tpu_sc as plsc

N = 50000
E = 800000
IN_F = 69
F1 = 96
CF1 = 24
HID = 128
OUT = 64
CF2 = 16

NC = 2
NT = 16

EPR = 80
E_PAD = 819200
ROWS = E_PAD // EPR
RPT = ROWS // NT
IDXR = 128
KG = 8
N_OUTER = RPT // IDXR
N_INNER = IDXR // KG

NAGG = 51200
STRIPE = NAGG // NT
ZR = 128
NZ = STRIPE // ZR

RB = 400
GRID = N // RB


def _deg_pass(srcd2d, dst2d):
    mesh = plsc.VectorSubcoreMesh(core_axis_name="c", subcore_axis_name="s")

    @functools.partial(
        pl.kernel,
        out_type=jax.ShapeDtypeStruct((NC, NAGG), jnp.float32),
        mesh=mesh,
        compiler_params=pltpu.CompilerParams(use_tc_tiling_on_sc=False),
        scratch_types=[
            pltpu.VMEM((IDXR, EPR), jnp.int32),
            pltpu.VMEM((EPR,), jnp.float32),
            pltpu.VMEM((STRIPE,), jnp.float32),
            pltpu.VMEM_SHARED((NAGG,), jnp.float32),
            pltpu.SemaphoreType.DMA,
        ],
    )
    def k(src_h, dst_h, out_h, idxv, ones_v, zflat, deg_sh, sem):
        c = lax.axis_index("c")
        s = lax.axis_index("s")

        zero16 = jnp.zeros((16,), jnp.float32)
        one16 = jnp.ones((16,), jnp.float32)

        def zfill(i, _):
            zflat[pl.ds(i * 16, 16)] = zero16
            return 0

        lax.fori_loop(0, STRIPE // 16, zfill, 0)
        for b in range(EPR // 16):
            ones_v[pl.ds(b * 16, 16)] = one16
        pltpu.sync_copy(zflat, deg_sh.at[pl.ds(s * STRIPE, STRIPE)])
        plsc.subcore_barrier()

        def process(idx_h):
            def outer(o, _):
                r0 = s * RPT + o * IDXR
                pltpu.sync_copy(idx_h.at[pl.ds(r0, IDXR)], idxv)

                def inner(g, _):
                    descs = [
                        pltpu.async_copy(
                            ones_v, deg_sh.at[idxv.at[g * KG + b]], sem, add=True
                        )
                        for b in range(KG)
                    ]
                    for d in descs:
                        d.wait()
                    return 0

                lax.fori_loop(0, N_INNER, inner, 0)
                return 0

            lax.fori_loop(0, N_OUTER, outer, 0)

        @pl.when(c == 0)
        def _():
            process(src_h)

        @pl.when(c == 1)
        def _():
            process(dst_h)

        plsc.subcore_barrier()
        pltpu.sync_copy(
            deg_sh.at[pl.ds(s * STRIPE, STRIPE)], out_h.at[c, pl.ds(s * STRIPE, STRIPE)]
        )

    return k(srcd2d, dst2d)


def _edge_pass(tabs, src2d, dst2d, zrows, F):
    mesh = plsc.VectorSubcoreMesh(core_axis_name="c", subcore_axis_name="s")

    @functools.partial(
        pl.kernel,
        out_type=jax.ShapeDtypeStruct((4, NAGG, F), jnp.float32),
        mesh=mesh,
        compiler_params=pltpu.CompilerParams(use_tc_tiling_on_sc=False),
        scratch_types=[
            pltpu.VMEM((IDXR, EPR), jnp.int32),
            pltpu.VMEM((IDXR, EPR), jnp.int32),
            pltpu.VMEM((KG, EPR, F), jnp.float32),
            pltpu.VMEM((KG, EPR, F), jnp.float32),
            pltpu.VMEM((ZR, F), jnp.float32),
            pltpu.VMEM_SHARED((NAGG, F), jnp.float32),
            pltpu.SemaphoreType.DMA,
            pltpu.SemaphoreType.DMA,
            pltpu.SemaphoreType.DMA,
            pltpu.SemaphoreType.DMA,
        ],
    )
    def k(t0_h, t1_h, t2_h, t3_h, src_h, dst_h, z_h, out_h,
          sidx, didx, rows_a, rows_b, zbuf, agg, gsem_a, gsem_b, ssem_a, ssem_b):
        c = lax.axis_index("c")
        s = lax.axis_index("s")

        pltpu.sync_copy(z_h, zbuf)

        def zero_stripe():
            def zloop(i, _):
                pltpu.sync_copy(zbuf, agg.at[pl.ds(s * STRIPE + i * ZR, ZR)])
                return 0

            lax.fori_loop(0, NZ, zloop, 0)

        NPAIR = N_INNER // 2

        def process(tab_h):
            def fire_g(g, rows, gsem):
                for b in range(KG):
                    pltpu.async_copy(tab_h.at[sidx.at[g * KG + b]], rows.at[b], gsem)

            def fire_s(g, rows, ssem):
                for b in range(KG):
                    pltpu.async_copy(
                        rows.at[b], agg.at[didx.at[g * KG + b]], ssem, add=True
                    )

            def wait_g(g, rows, gsem):
                for b in range(KG):
                    pltpu.make_async_copy(
                        tab_h.at[sidx.at[g * KG + b]], rows.at[b], gsem
                    ).wait()

            def wait_s(g, rows, ssem):
                for b in range(KG):
                    pltpu.make_async_copy(
                        rows.at[b], agg.at[didx.at[g * KG + b]], ssem
                    ).wait()

            def outer(o, _):
                r0 = s * RPT + o * IDXR
                pltpu.sync_copy(src_h.at[pl.ds(r0, IDXR)], sidx)
                pltpu.sync_copy(dst_h.at[pl.ds(r0, IDXR)], didx)

                fire_g(0, rows_a, gsem_a)
                wait_g(0, rows_a, gsem_a)
                fire_g(1, rows_b, gsem_b)
                fire_s(0, rows_a, ssem_a)

                def inner(gg, _):
                    g0 = 2 * gg
                    wait_g(g0 - 1, rows_b, gsem_b)
                    wait_s(g0 - 2, rows_a, ssem_a)
                    fire_g(g0, rows_a, gsem_a)
                    fire_s(g0 - 1, rows_b, ssem_b)
                    wait_g(g0, rows_a, gsem_a)
                    wait_s(g0 - 1, rows_b, ssem_b)
                    fire_g(g0 + 1, rows_b, gsem_b)
                    fire_s(g0, rows_a, ssem_a)
                    return 0

                lax.fori_loop(1, NPAIR, inner, 0)
                g_last = 2 * NPAIR - 1
                wait_g(g_last, rows_b, gsem_b)
                wait_s(g_last - 1, rows_a, ssem_a)
                fire_s(g_last, rows_b, ssem_b)
                wait_s(g_last, rows_b, ssem_b)
                return 0

            lax.fori_loop(0, N_OUTER, outer, 0)

        def writeout(q):
            def wloop(i, _):
                off = s * STRIPE + i * ZR
                pltpu.sync_copy(agg.at[pl.ds(off, ZR)], out_h.at[q, pl.ds(off, ZR)])
                return 0

            lax.fori_loop(0, NZ, wloop, 0)

        zero_stripe()
        plsc.subcore_barrier()

        @pl.when(c == 0)
        def _():
            process(t0_h)

        @pl.when(c == 1)
        def _():
            process(t2_h)

        plsc.subcore_barrier()
        writeout(2 * c)
        zero_stripe()
        plsc.subcore_barrier()

        @pl.when(c == 0)
        def _():
            process(t1_h)

        @pl.when(c == 1)
        def _():
            process(t3_h)

        plsc.subcore_barrier()
        writeout(2 * c + 1)

    return k(tabs[0], tabs[1], tabs[2], tabs[3], src2d, dst2d, zrows)


def _k2_body(deg_ref, x0_ref, x1_ref, x2_ref, x3_ref,
             o0_ref, o1_ref, o2_ref, o3_ref, ns_ref, nd_ref):
    d = deg_ref[...]
    ns = lax.rsqrt(jnp.maximum(d[:, 0:1], 1.0))
    nd = lax.rsqrt(jnp.maximum(d[:, 1:2], 1.0))
    o0_ref[...] = x0_ref[...] * ns
    o1_ref[...] = x1_ref[...] * ns
    o2_ref[...] = x2_ref[...] * ns
    o3_ref[...] = x3_ref[...] * ns
    ns_ref[...] = ns
    nd_ref[...] = nd


def _k2(deg_t, xqs):
    xspec = pl.BlockSpec((RB, CF1), lambda i: (i, 0))
    nspec = pl.BlockSpec((RB, 1), lambda i: (i, 0))
    return pl.pallas_call(
        _k2_body,
        grid=(GRID,),
        in_specs=[pl.BlockSpec((RB, NC), lambda i: (i, 0))] + [xspec] * 4,
        out_specs=[xspec] * 4 + [nspec, nspec],
        out_shape=[jax.ShapeDtypeStruct((N, CF1), jnp.float32)] * 4
        + [jax.ShapeDtypeStruct((N, 1), jnp.float32)] * 2,
    )(deg_t, *xqs)


def _k4_body(a0_ref, a1_ref, a2_ref, a3_ref, ns_ref, nd_ref,
             w10_ref, w11_ref, w12_ref, w13_ref,
             w20_ref, w21_ref, w22_ref, w23_ref,
             p0_ref, p1_ref, p2_ref, p3_ref):
    nd = nd_ref[...]
    z = jnp.dot(a0_ref[0] * nd, w10_ref[...], preferred_element_type=jnp.float32)
    z = z + jnp.dot(a1_ref[0] * nd, w11_ref[...], preferred_element_type=jnp.float32)
    z = z + jnp.dot(a2_ref[0] * nd, w12_ref[...], preferred_element_type=jnp.float32)
    z = z + jnp.dot(a3_ref[0] * nd, w13_ref[...], preferred_element_type=jnp.float32)
    z = jnp.maximum(z, 0.0) * ns_ref[...]
    p0_ref[...] = jnp.dot(z, w20_ref[...], preferred_element_type=jnp.float32)
    p1_ref[...] = jnp.dot(z, w21_ref[...], preferred_element_type=jnp.float32)
    p2_ref[...] = jnp.dot(z, w22_ref[...], preferred_element_type=jnp.float32)
    p3_ref[...] = jnp.dot(z, w23_ref[...], preferred_element_type=jnp.float32)


def _k4(agg1, ns, nd, w1qs, w2qs):
    aspec = [
        pl.BlockSpec((1, RB, CF1), (lambda q: (lambda i: (q, i, 0)))(q))
        for q in range(4)
    ]
    nspec = pl.BlockSpec((RB, 1), lambda i: (i, 0))
    return pl.pallas_call(
        _k4_body,
        grid=(GRID,),
        in_specs=aspec
        + [nspec, nspec]
        + [pl.BlockSpec((CF1, HID), lambda i: (0, 0))] * 4
        + [pl.BlockSpec((HID, CF2), lambda i: (0, 0))] * 4,
        out_specs=[pl.BlockSpec((RB, CF2), lambda i: (i, 0))] * 4,
        out_shape=[jax.ShapeDtypeStruct((N, CF2), jnp.float32)] * 4,
    )(agg1, agg1, agg1, agg1, ns, nd, *w1qs, *w2qs)


def _k6_body(a0_ref, a1_ref, a2_ref, a3_ref, nd_ref, o_ref):
    i = pl.program_id(0)
    nd = nd_ref[...]
    ms = [
        jnp.max(a_ref[0] * nd, axis=0, keepdims=True)
        for a_ref in (a0_ref, a1_ref, a2_ref, a3_ref)
    ]
    m = jnp.concatenate(ms, axis=0)

    @pl.when(i == 0)
    def _():
        o_ref[...] = m

    @pl.when(i > 0)
    def _():
        o_ref[...] = jnp.maximum(o_ref[...], m)

    @pl.when(i == GRID - 1)
    def _():
        o_ref[...] = jnp.maximum(o_ref[...], 0.0)


def _k6(agg2, nd):
    aspec = [
        pl.BlockSpec((1, RB, CF2), (lambda q: (lambda i: (q, i, 0)))(q))
        for q in range(4)
    ]
    return pl.pallas_call(
        _k6_body,
        grid=(GRID,),
        in_specs=aspec + [pl.BlockSpec((RB, 1), lambda i: (i, 0))],
        out_specs=pl.BlockSpec((4, CF2), lambda i: (0, 0)),
        out_shape=jax.ShapeDtypeStruct((4, CF2), jnp.float32),
    )(agg2, agg2, agg2, agg2, nd)


def kernel(x, edge_index, W1, W2):
    ei = edge_index.astype(jnp.int32)
    src = ei[0]
    dst = ei[1]

    npad = E_PAD - E
    ar = jnp.arange(npad, dtype=jnp.int32)
    trash = N + (ar % (NAGG - N - 8))
    src_deg2d = jnp.concatenate([src, trash]).reshape(ROWS, EPR)
    src_edge2d = jnp.concatenate([src, ar % 1024]).reshape(ROWS, EPR)
    dst2d = jnp.concatenate([dst, trash]).reshape(ROWS, EPR)

    xp = jnp.pad(x, ((0, 0), (0, F1 - IN_F)))
    xqs = [xp[:, q * CF1:(q + 1) * CF1] for q in range(4)]
    w1p = jnp.pad(W1, ((0, F1 - IN_F), (0, 0)))
    w1qs = [w1p[q * CF1:(q + 1) * CF1] for q in range(4)]
    w2qs = [W2[:, q * CF2:(q + 1) * CF2] for q in range(4)]
    z24 = jnp.zeros((ZR, CF1), jnp.float32)
    z16 = jnp.zeros((ZR, CF2), jnp.float32)

    deg = _deg_pass(src_deg2d, dst2d)
    deg_t = deg[:, :N].T
    k2out = _k2(deg_t, xqs)
    xn_qs, ns, nd = k2out[:4], k2out[4], k2out[5]
    agg1 = _edge_pass(xn_qs, src_edge2d, dst2d, z24, CF1)[:, :N]
    pqs = _k4(agg1, ns, nd, w1qs, w2qs)
    agg2 = _edge_pass(pqs, src_edge2d, dst2d, z16, CF2)[:, :N]
    out = _k6(agg2, nd)
    return out.reshape(1, OUT)


# speedup vs baseline: 6.5471x; 1.0228x over previous
import functools

import jax
import jax.numpy as jnp
from jax import lax
from jax.experimental import pallas as pl
from jax.experimental.pallas import tpu as pltpu
from jax.experimental.pallas import tpu_sc as plsc

N = 50000
E = 800000
IN_F = 69
F1 = 96
HID = 128
OUT = 64

NC = 2
NT = 16

EPR = 128
E_PAD = 819200
ROWS = E_PAD // EPR
RPT = ROWS // NT
IDXR = 80
KG = 8
N_OUTER = RPT // IDXR
N_INNER = IDXR // KG

NAGG = 51200
RNG = 4
RROWS = NAGG // RNG
RTRASH = 8
RROWS_T = RROWS + RTRASH

RINGSZ = 11264
CAP = 224 * 256
PSTG = 16

STRIPE = NAGG // NT
RSTRIPE = RROWS // NT
WZR = 160
NWZ = RSTRIPE // WZR

RB = 400
GRID2 = NAGG // RB
GRID = N // RB


def _deg_pass(srcd2d, dst2d):
    mesh = plsc.VectorSubcoreMesh(core_axis_name="c", subcore_axis_name="s")

    @functools.partial(
        pl.kernel,
        out_type=jax.ShapeDtypeStruct((NC, NAGG), jnp.float32),
        mesh=mesh,
        compiler_params=pltpu.CompilerParams(use_tc_tiling_on_sc=False),
        scratch_types=[
            pltpu.VMEM((IDXR, EPR), jnp.int32),
            pltpu.VMEM((EPR,), jnp.float32),
            pltpu.VMEM((STRIPE,), jnp.float32),
            pltpu.VMEM_SHARED((NAGG,), jnp.float32),
            pltpu.SemaphoreType.DMA,
        ],
    )
    def k(src_h, dst_h, out_h, idxv, ones_v, zflat, deg_sh, sem):
        c = lax.axis_index("c")
        s = lax.axis_index("s")

        zero16 = jnp.zeros((16,), jnp.float32)
        one16 = jnp.ones((16,), jnp.float32)

        def zfill(i, _):
            zflat[pl.ds(i * 16, 16)] = zero16
            return 0

        lax.fori_loop(0, STRIPE // 16, zfill, 0)
        for b in range(EPR // 16):
            ones_v[pl.ds(b * 16, 16)] = one16
        pltpu.sync_copy(zflat, deg_sh.at[pl.ds(s * STRIPE, STRIPE)])
        plsc.subcore_barrier()

        def process(idx_h):
            def outer(o, _):
                r0 = s * RPT + o * IDXR
                pltpu.sync_copy(idx_h.at[pl.ds(r0, IDXR)], idxv)

                def inner(g, _):
                    descs = [
                        pltpu.async_copy(
                            ones_v, deg_sh.at[idxv.at[g * KG + b]], sem, add=True
                        )
                        for b in range(KG)
                    ]
                    for d in descs:
                        d.wait()
                    return 0

                lax.fori_loop(0, N_INNER, inner, 0)
                return 0

            lax.fori_loop(0, N_OUTER, outer, 0)

        @pl.when(c == 0)
        def _():
            process(src_h)

        @pl.when(c == 1)
        def _():
            process(dst_h)

        plsc.subcore_barrier()
        pltpu.sync_copy(
            deg_sh.at[pl.ds(s * STRIPE, STRIPE)], out_h.at[c, pl.ds(s * STRIPE, STRIPE)]
        )

    return k(srcd2d, dst2d)


def _partition(src2d, dst2d):
    mesh = plsc.VectorSubcoreMesh(core_axis_name="c", subcore_axis_name="s")

    @functools.partial(
        pl.kernel,
        out_type=[
            jax.ShapeDtypeStruct((NC, NT, 2, CAP), jnp.int32),
            jax.ShapeDtypeStruct((NC, NT, 2, 16), jnp.int32),
        ],
        mesh=mesh,
        compiler_params=pltpu.CompilerParams(
            use_tc_tiling_on_sc=False, needs_layout_passes=False),
        scratch_types=[
            pltpu.VMEM((IDXR, EPR), jnp.int32),
            pltpu.VMEM((IDXR, EPR), jnp.int32),
            pltpu.VMEM((RINGSZ,), jnp.int32),
            pltpu.VMEM((RINGSZ,), jnp.int32),
            pltpu.VMEM((2, 16), jnp.int32),
        ],
    )
    def k(src_h, dst_h, plist_h, pcnt_h, sidx, didx, ring0, ring1, cntv):
        c = lax.axis_index("c")
        s = lax.axis_index("s")
        lo0 = (2 * c) * RROWS
        lo0s = lo0 * 65536
        los = RROWS * 65536
        tmask = jnp.ones((16,), jnp.bool_)
        trash16 = jnp.full((16,), RROWS * 65536, jnp.int32)

        rings = (ring0, ring1)

        def count(m):
            return jnp.max(plsc.all_reduce_population_count(m))

        def flush(ring, j, pos, fl):
            full = pos // 256

            def fk(kk, _):
                pltpu.sync_copy(
                    ring.at[pl.ds(kk * 256, 256)],
                    plist_h.at[c, s, j, pl.ds((fl + kk) * 256, 256)],
                )
                return 0

            lax.fori_loop(0, full, fk, 0)

            @pl.when(full > 0)
            def _():
                for t in range(16):
                    v = ring[pl.ds(full * 256 + 16 * t, 16)]
                    ring[pl.ds(16 * t, 16)] = v

            return pos - full * 256, fl + full

        def outer(o, carry):
            pos0, fl0, pos1, fl1 = carry
            r0 = s * RPT + o * IDXR
            pltpu.sync_copy(src_h.at[pl.ds(r0, IDXR)], sidx)
            pltpu.sync_copy(dst_h.at[pl.ds(r0, IDXR)], didx)

            def crow(r, carry2):
                pos0, pos1 = carry2
                for m in range(EPR // 16):
                    sv = sidx[r, pl.ds(m * 16, 16)]
                    dv = didx[r, pl.ds(m * 16, 16)]
                    dvs = dv * 65536
                    m0 = (dv >= lo0) & (dv < lo0 + RROWS)
                    p0 = (dvs - lo0s) | sv
                    plsc.store_compressed(ring0.at[pl.ds(pos0, 16)], p0, mask=m0)
                    pos0 = pos0 + count(m0)
                    m1 = (dv >= lo0 + RROWS) & (dv < lo0 + 2 * RROWS)
                    p1 = (dvs - lo0s - los) | sv
                    plsc.store_compressed(ring1.at[pl.ds(pos1, 16)], p1, mask=m1)
                    pos1 = pos1 + count(m1)
                return (pos0, pos1)

            pos0, pos1 = lax.fori_loop(0, IDXR, crow, (pos0, pos1))
            pos0, fl0 = flush(ring0, 0, pos0, fl0)
            pos1, fl1 = flush(ring1, 1, pos1, fl1)
            return (pos0, fl0, pos1, fl1)

        z = jnp.int32(0)
        pos0, fl0, pos1, fl1 = lax.fori_loop(0, N_OUTER, outer, (z, z, z, z))

        for j, (ring, pos, fl) in enumerate(((ring0, pos0, fl0), (ring1, pos1, fl1))):
            for t in range(16):
                plsc.store_compressed(ring.at[pl.ds(pos + 16 * t, 16)], trash16, mask=tmask)

            @pl.when(pos > 0)
            def _():
                pltpu.sync_copy(
                    ring.at[pl.ds(0, 256)],
                    plist_h.at[c, s, j, pl.ds(fl * 256, 256)],
                )

            n2 = fl + jnp.minimum(pos, 1)
            cntv[j, :] = jnp.full((16,), 1, jnp.int32) * n2

        pltpu.sync_copy(cntv, pcnt_h.at[c, s])

    return k(src2d, dst2d)


def _edge_pass(tab, plist, pcnt, zrows, F):
    mesh = plsc.VectorSubcoreMesh(core_axis_name="c", subcore_axis_name="s")

    @functools.partial(
        pl.kernel,
        out_type=jax.ShapeDtypeStruct((RNG, RROWS, F), jnp.float32),
        mesh=mesh,
        compiler_params=pltpu.CompilerParams(use_tc_tiling_on_sc=False),
        scratch_types=[
            pltpu.VMEM((PSTG * 256,), jnp.int32),
            pltpu.VMEM((2, EPR), jnp.int32),
            pltpu.VMEM((2, EPR), jnp.int32),
            pltpu.VMEM((EPR, F), jnp.float32),
            pltpu.VMEM((EPR, F), jnp.float32),
            pltpu.VMEM((WZR, F), jnp.float32),
            pltpu.VMEM((2, 16), jnp.int32),
            pltpu.VMEM_SHARED((RROWS_T, F), jnp.float32),
            pltpu.SemaphoreType.DMA,
            pltpu.SemaphoreType.DMA,
            pltpu.SemaphoreType.DMA,
            pltpu.SemaphoreType.DMA,
        ],
    )
    def k(tab_h, plist_h, pcnt_h, z_h, out_h,
          pbuf, sidxb, didxb, rows_a, rows_b, zbuf, cntv, agg,
          gsa, gsb, ssa, ssb):
        c = lax.axis_index("c")
        s = lax.axis_index("s")

        pltpu.sync_copy(z_h, zbuf)
        pltpu.sync_copy(pcnt_h.at[c, s], cntv)

        def zero_stripe():
            def zloop(i, _):
                pltpu.sync_copy(zbuf, agg.at[pl.ds(s * RSTRIPE + i * WZR, WZR)])
                return 0

            lax.fori_loop(0, NWZ, zloop, 0)

            @pl.when(s == 0)
            def _():
                pltpu.sync_copy(zbuf.at[pl.ds(0, RTRASH)], agg.at[pl.ds(RROWS, RTRASH)])

        def writeout(q):
            def wloop(i, _):
                off = s * RSTRIPE + i * WZR
                pltpu.sync_copy(agg.at[pl.ds(off, WZR)], out_h.at[q, pl.ds(off, WZR)])
                return 0

            lax.fori_loop(0, NWZ, wloop, 0)

        def process(r):
            n2 = cntv[r, pl.ds(0, 16)][0]
            nst = (n2 + PSTG - 1) // PSTG

            def souter(t, _):
                base = t * PSTG
                mm = jnp.minimum(PSTG, n2 - base)
                pltpu.sync_copy(
                    plist_h.at[c, s, r, pl.ds(base * 256, PSTG * 256)], pbuf
                )

                def pair(jp, _):
                    for mchunk in range(16):
                        pk = pbuf[pl.ds(jp * 256 + mchunk * 16, 16)]
                        sv = pk & 0xFFFF
                        dv = lax.shift_right_logical(pk, 16)
                        row = mchunk // 8
                        col = (mchunk % 8) * 16
                        sidxb[row, pl.ds(col, 16)] = sv
                        didxb[row, pl.ds(col, 16)] = dv
                    ga = pltpu.async_copy(tab_h.at[sidxb.at[0]], rows_a, gsa)
                    gb = pltpu.async_copy(tab_h.at[sidxb.at[1]], rows_b, gsb)
                    ga.wait()
                    sa = pltpu.async_copy(rows_a, agg.at[didxb.at[0]], ssa, add=True)
                    gb.wait()
                    sb = pltpu.async_copy(rows_b, agg.at[didxb.at[1]], ssb, add=True)
                    sa.wait()
                    sb.wait()
                    return 0

                lax.fori_loop(0, mm, pair, 0)
                return 0

            lax.fori_loop(0, nst, souter, 0)

        zero_stripe()
        plsc.subcore_barrier()
        process(0)
        plsc.subcore_barrier()
        writeout(2 * c)
        zero_stripe()
        plsc.subcore_barrier()
        process(1)
        plsc.subcore_barrier()
        writeout(2 * c + 1)

    return k(tab, plist, pcnt, zrows)


def _k2_body(deg_ref, x_ref, o_ref, ns_ref, nd_ref):
    d = deg_ref[...]
    ns = lax.rsqrt(jnp.maximum(d[:, 0:1], 1.0))
    nd = lax.rsqrt(jnp.maximum(d[:, 1:2], 1.0))
    o_ref[...] = x_ref[...] * ns
    ns_ref[...] = ns
    nd_ref[...] = nd


def _k2(deg_t, xp):
    return pl.pallas_call(
        _k2_body,
        grid=(GRID2,),
        in_specs=[
            pl.BlockSpec((RB, NC), lambda i: (i, 0)),
            pl.BlockSpec((RB, F1), lambda i: (i, 0)),
        ],
        out_specs=[
            pl.BlockSpec((RB, F1), lambda i: (i, 0)),
            pl.BlockSpec((RB, 1), lambda i: (i, 0)),
            pl.BlockSpec((RB, 1), lambda i: (i, 0)),
        ],
        out_shape=[
            jax.ShapeDtypeStruct((NAGG, F1), jnp.float32),
            jax.ShapeDtypeStruct((NAGG, 1), jnp.float32),
            jax.ShapeDtypeStruct((NAGG, 1), jnp.float32),
        ],
    )(deg_t, xp)


def _k4_body(a_ref, ns_ref, nd_ref, w1_ref, w2_ref, p_ref):
    z = jnp.dot(a_ref[...] * nd_ref[...], w1_ref[...],
                preferred_element_type=jnp.float32)
    z = jnp.maximum(z, 0.0) * ns_ref[...]
    p_ref[...] = jnp.dot(z, w2_ref[...], preferred_element_type=jnp.float32)


def _k4(agg1, ns, nd, w1p, W2):
    return pl.pallas_call(
        _k4_body,
        grid=(GRID2,),
        in_specs=[
            pl.BlockSpec((RB, F1), lambda i: (i, 0)),
            pl.BlockSpec((RB, 1), lambda i: (i, 0)),
            pl.BlockSpec((RB, 1), lambda i: (i, 0)),
            pl.BlockSpec((F1, HID), lambda i: (0, 0)),
            pl.BlockSpec((HID, OUT), lambda i: (0, 0)),
        ],
        out_specs=pl.BlockSpec((RB, OUT), lambda i: (i, 0)),
        out_shape=jax.ShapeDtypeStruct((NAGG, OUT), jnp.float32),
    )(agg1, ns, nd, w1p, W2)


def _k6_body(a_ref, nd_ref, o_ref):
    i = pl.program_id(0)
    m = jnp.max(a_ref[...] * nd_ref[...], axis=0, keepdims=True)

    @pl.when(i == 0)
    def _():
        o_ref[...] = m

    @pl.when(i > 0)
    def _():
        o_ref[...] = jnp.maximum(o_ref[...], m)

    @pl.when(i == GRID - 1)
    def _():
        o_ref[...] = jnp.maximum(o_ref[...], 0.0)


def _k6(agg2, nd):
    return pl.pallas_call(
        _k6_body,
        grid=(GRID,),
        in_specs=[
            pl.BlockSpec((RB, OUT), lambda i: (i, 0)),
            pl.BlockSpec((RB, 1), lambda i: (i, 0)),
        ],
        out_specs=pl.BlockSpec((1, OUT), lambda i: (0, 0)),
        out_shape=jax.ShapeDtypeStruct((1, OUT), jnp.float32),
    )(agg2, nd)


def kernel(x, edge_index, W1, W2):
    ei = edge_index.astype(jnp.int32)
    src = ei[0]
    dst = ei[1]

    npad = E_PAD - E
    ar = jnp.arange(npad, dtype=jnp.int32)
    trash = N + (ar % 1024)
    src_deg2d = jnp.concatenate([src, trash]).reshape(ROWS, EPR)
    src_edge2d = jnp.concatenate([src, ar % 1024]).reshape(ROWS, EPR)
    dst2d = jnp.concatenate([dst, trash]).reshape(ROWS, EPR)

    xp = jnp.pad(x, ((0, NAGG - N), (0, F1 - IN_F)))
    w1p = jnp.pad(W1, ((0, F1 - IN_F), (0, 0)))
    z96 = jnp.zeros((WZR, F1), jnp.float32)
    z64 = jnp.zeros((WZR, OUT), jnp.float32)

    deg = _deg_pass(src_deg2d, dst2d)
    deg_t = deg.T
    plist, pcnt = _partition(src_edge2d, dst2d)
    xn, ns, nd = _k2(deg_t, xp)
    agg1 = _edge_pass(xn, plist, pcnt, z96, F1).reshape(NAGG, F1)
    p = _k4(agg1, ns, nd, w1p, W2)
    agg2 = _edge_pass(p, plist, pcnt, z64, OUT).reshape(NAGG, OUT)
    return _k6(agg2, nd)


# speedup vs baseline: 7.2977x; 1.1146x over previous
import functools

import jax
import jax.numpy as jnp
from jax import lax
from jax.experimental import pallas as pl
from jax.experimental.pallas import tpu as pltpu
from jax.experimental.pallas import tpu_sc as plsc

N = 50000
E = 800000
IN_F = 69
F1 = 96
HID = 128
OUT = 64

NC = 2
NT = 16

EPR = 128
E_PAD = 819200
ROWS = E_PAD // EPR
RPT = ROWS // NT
IDXR = 80
KG = 8
N_OUTER = RPT // IDXR
N_INNER = IDXR // KG

NAGG = 51200
RNG = 4
RROWS = NAGG // RNG
RTRASH = 8
RROWS_T = RROWS + RTRASH

RINGSZ = 11264
CAP = 224 * 256
PSTG = 16

STRIPE = NAGG // NT
RSTRIPE = RROWS // NT
WZR = 160
NWZ = RSTRIPE // WZR

RB = 400
GRID2 = NAGG // RB
GRID = N // RB


def _deg_pass(srcd2d, dst2d):
    mesh = plsc.VectorSubcoreMesh(core_axis_name="c", subcore_axis_name="s")

    @functools.partial(
        pl.kernel,
        out_type=jax.ShapeDtypeStruct((NC, NAGG), jnp.float32),
        mesh=mesh,
        compiler_params=pltpu.CompilerParams(use_tc_tiling_on_sc=False),
        scratch_types=[
            pltpu.VMEM((IDXR, EPR), jnp.int32),
            pltpu.VMEM((EPR,), jnp.float32),
            pltpu.VMEM((STRIPE,), jnp.float32),
            pltpu.VMEM_SHARED((NAGG,), jnp.float32),
            pltpu.SemaphoreType.DMA,
        ],
    )
    def k(src_h, dst_h, out_h, idxv, ones_v, zflat, deg_sh, sem):
        c = lax.axis_index("c")
        s = lax.axis_index("s")

        zero16 = jnp.zeros((16,), jnp.float32)
        one16 = jnp.ones((16,), jnp.float32)

        def zfill(i, _):
            zflat[pl.ds(i * 16, 16)] = zero16
            return 0

        lax.fori_loop(0, STRIPE // 16, zfill, 0)
        for b in range(EPR // 16):
            ones_v[pl.ds(b * 16, 16)] = one16
        pltpu.sync_copy(zflat, deg_sh.at[pl.ds(s * STRIPE, STRIPE)])
        plsc.subcore_barrier()

        def process(idx_h):
            def outer(o, _):
                r0 = s * RPT + o * IDXR
                pltpu.sync_copy(idx_h.at[pl.ds(r0, IDXR)], idxv)

                def inner(g, _):
                    descs = [
                        pltpu.async_copy(
                            ones_v, deg_sh.at[idxv.at[g * KG + b]], sem, add=True
                        )
                        for b in range(KG)
                    ]
                    for d in descs:
                        d.wait()
                    return 0

                lax.fori_loop(0, N_INNER, inner, 0)
                return 0

            lax.fori_loop(0, N_OUTER, outer, 0)

        @pl.when(c == 0)
        def _():
            process(src_h)

        @pl.when(c == 1)
        def _():
            process(dst_h)

        plsc.subcore_barrier()
        pltpu.sync_copy(
            deg_sh.at[pl.ds(s * STRIPE, STRIPE)], out_h.at[c, pl.ds(s * STRIPE, STRIPE)]
        )

    return k(srcd2d, dst2d)


def _partition(src2d, dst2d):
    mesh = plsc.VectorSubcoreMesh(core_axis_name="c", subcore_axis_name="s")

    @functools.partial(
        pl.kernel,
        out_type=[
            jax.ShapeDtypeStruct((NC, NT, 2, CAP), jnp.int32),
            jax.ShapeDtypeStruct((NC, NT, 2, 16), jnp.int32),
        ],
        mesh=mesh,
        compiler_params=pltpu.CompilerParams(
            use_tc_tiling_on_sc=False, needs_layout_passes=False),
        scratch_types=[
            pltpu.VMEM((IDXR, EPR), jnp.int32),
            pltpu.VMEM((IDXR, EPR), jnp.int32),
            pltpu.VMEM((RINGSZ,), jnp.int32),
            pltpu.VMEM((RINGSZ,), jnp.int32),
            pltpu.VMEM((2, 16), jnp.int32),
        ],
    )
    def k(src_h, dst_h, plist_h, pcnt_h, sidx, didx, ring0, ring1, cntv):
        c = lax.axis_index("c")
        s = lax.axis_index("s")
        lo0 = (2 * c) * RROWS
        lo0s = lo0 * 65536
        los = RROWS * 65536
        tmask = jnp.ones((16,), jnp.bool_)
        trash16 = jnp.full((16,), RROWS * 65536, jnp.int32)

        rings = (ring0, ring1)

        def count(m):
            return jnp.max(plsc.all_reduce_population_count(m))

        def flush(ring, j, pos, fl):
            full = pos // 256

            def fk(kk, _):
                pltpu.sync_copy(
                    ring.at[pl.ds(kk * 256, 256)],
                    plist_h.at[c, s, j, pl.ds((fl + kk) * 256, 256)],
                )
                return 0

            lax.fori_loop(0, full, fk, 0)

            @pl.when(full > 0)
            def _():
                for t in range(16):
                    v = ring[pl.ds(full * 256 + 16 * t, 16)]
                    ring[pl.ds(16 * t, 16)] = v

            return pos - full * 256, fl + full

        def outer(o, carry):
            pos0, fl0, pos1, fl1 = carry
            r0 = s * RPT + o * IDXR
            pltpu.sync_copy(src_h.at[pl.ds(r0, IDXR)], sidx)
            pltpu.sync_copy(dst_h.at[pl.ds(r0, IDXR)], didx)

            def crow(r, carry2):
                pos0, pos1 = carry2
                for m in range(EPR // 16):
                    sv = sidx[r, pl.ds(m * 16, 16)]
                    dv = didx[r, pl.ds(m * 16, 16)]
                    dvs = dv * 65536
                    m0 = (dv >= lo0) & (dv < lo0 + RROWS)
                    p0 = (dvs - lo0s) | sv
                    plsc.store_compressed(ring0.at[pl.ds(pos0, 16)], p0, mask=m0)
                    pos0 = pos0 + count(m0)
                    m1 = (dv >= lo0 + RROWS) & (dv < lo0 + 2 * RROWS)
                    p1 = (dvs - lo0s - los) | sv
                    plsc.store_compressed(ring1.at[pl.ds(pos1, 16)], p1, mask=m1)
                    pos1 = pos1 + count(m1)
                return (pos0, pos1)

            pos0, pos1 = lax.fori_loop(0, IDXR, crow, (pos0, pos1))
            pos0, fl0 = flush(ring0, 0, pos0, fl0)
            pos1, fl1 = flush(ring1, 1, pos1, fl1)
            return (pos0, fl0, pos1, fl1)

        z = jnp.int32(0)
        pos0, fl0, pos1, fl1 = lax.fori_loop(0, N_OUTER, outer, (z, z, z, z))

        for j, (ring, pos, fl) in enumerate(((ring0, pos0, fl0), (ring1, pos1, fl1))):
            for t in range(16):
                plsc.store_compressed(ring.at[pl.ds(pos + 16 * t, 16)], trash16, mask=tmask)

            @pl.when(pos > 0)
            def _():
                pltpu.sync_copy(
                    ring.at[pl.ds(0, 256)],
                    plist_h.at[c, s, j, pl.ds(fl * 256, 256)],
                )

            n2 = fl + jnp.minimum(pos, 1)
            cntv[j, :] = jnp.full((16,), 1, jnp.int32) * n2

        pltpu.sync_copy(cntv, pcnt_h.at[c, s])

    return k(src2d, dst2d)


def _edge_pass(tab, plist, pcnt, zrows, F):
    mesh = plsc.VectorSubcoreMesh(core_axis_name="c", subcore_axis_name="s")

    @functools.partial(
        pl.kernel,
        out_type=jax.ShapeDtypeStruct((RNG, RROWS, F), jnp.float32),
        mesh=mesh,
        compiler_params=pltpu.CompilerParams(use_tc_tiling_on_sc=False),
        scratch_types=[
            pltpu.VMEM((PSTG * 256,), jnp.int32),
            pltpu.VMEM((2, EPR), jnp.int32),
            pltpu.VMEM((2, EPR), jnp.int32),
            pltpu.VMEM((EPR, F), jnp.float32),
            pltpu.VMEM((EPR, F), jnp.float32),
            pltpu.VMEM((WZR, F), jnp.float32),
            pltpu.VMEM((2, 16), jnp.int32),
            pltpu.VMEM_SHARED((RROWS_T, F), jnp.float32),
            pltpu.SemaphoreType.DMA,
            pltpu.SemaphoreType.DMA,
            pltpu.SemaphoreType.DMA,
            pltpu.SemaphoreType.DMA,
        ],
    )
    def k(tab_h, plist_h, pcnt_h, z_h, out_h,
          pbuf, sidxb, didxb, rows_a, rows_b, zbuf, cntv, agg,
          gsa, gsb, ssa, ssb):
        c = lax.axis_index("c")
        s = lax.axis_index("s")

        pltpu.sync_copy(z_h, zbuf)
        pltpu.sync_copy(pcnt_h.at[c, s], cntv)

        def zero_stripe():
            def zloop(i, _):
                pltpu.sync_copy(zbuf, agg.at[pl.ds(s * RSTRIPE + i * WZR, WZR)])
                return 0

            lax.fori_loop(0, NWZ, zloop, 0)

            @pl.when(s == 0)
            def _():
                pltpu.sync_copy(zbuf.at[pl.ds(0, RTRASH)], agg.at[pl.ds(RROWS, RTRASH)])

        def writeout(q):
            def wloop(i, _):
                off = s * RSTRIPE + i * WZR
                pltpu.sync_copy(agg.at[pl.ds(off, WZR)], out_h.at[q, pl.ds(off, WZR)])
                return 0

            lax.fori_loop(0, NWZ, wloop, 0)

        def unpack(b, P):
            for mc in range(8):
                pk = pbuf[pl.ds(b * 128 + mc * 16, 16)]
                sidxb[P, pl.ds(mc * 16, 16)] = pk & 0xFFFF
                didxb[P, pl.ds(mc * 16, 16)] = lax.shift_right_logical(pk, 16)

        def process(r):
            n2 = cntv[r, pl.ds(0, 16)][0]
            nfull = n2 // PSTG
            rows = (rows_a, rows_b)
            gsems = (gsa, gsb)
            ssems = (ssa, ssb)

            def souter(t, _):
                base = t * PSTG
                pltpu.sync_copy(
                    plist_h.at[c, s, r, pl.ds(base * 256, PSTG * 256)], pbuf
                )
                nb = 2 * PSTG
                gds = [None] * nb
                sds = [None] * nb
                for b in range(nb):
                    P = b & 1
                    if b >= 2:
                        sds[b - 2].wait()
                    unpack(b, P)
                    gds[b] = pltpu.async_copy(
                        tab_h.at[sidxb.at[P]], rows[P], gsems[P]
                    )
                    if b >= 1:
                        Q = (b - 1) & 1
                        gds[b - 1].wait()
                        sds[b - 1] = pltpu.async_copy(
                            rows[Q], agg.at[didxb.at[Q]], ssems[Q], add=True
                        )
                gds[nb - 1].wait()
                sds[nb - 1] = pltpu.async_copy(
                    rows[(nb - 1) & 1], agg.at[didxb.at[(nb - 1) & 1]],
                    ssems[(nb - 1) & 1], add=True
                )
                sds[nb - 2].wait()
                sds[nb - 1].wait()
                return 0

            lax.fori_loop(0, nfull, souter, 0)

            tbase = nfull * PSTG
            mm = n2 - tbase
            pltpu.sync_copy(
                plist_h.at[c, s, r, pl.ds(tbase * 256, PSTG * 256)], pbuf
            )

            def pair(jp, _):
                unpack(2 * jp, 0)
                unpack(2 * jp + 1, 1)
                ga = pltpu.async_copy(tab_h.at[sidxb.at[0]], rows_a, gsa)
                gb = pltpu.async_copy(tab_h.at[sidxb.at[1]], rows_b, gsb)
                ga.wait()
                sa = pltpu.async_copy(rows_a, agg.at[didxb.at[0]], ssa, add=True)
                gb.wait()
                sb = pltpu.async_copy(rows_b, agg.at[didxb.at[1]], ssb, add=True)
                sa.wait()
                sb.wait()
                return 0

            lax.fori_loop(0, mm, pair, 0)

        zero_stripe()
        plsc.subcore_barrier()
        process(0)
        plsc.subcore_barrier()
        writeout(2 * c)
        zero_stripe()
        plsc.subcore_barrier()
        process(1)
        plsc.subcore_barrier()
        writeout(2 * c + 1)

    return k(tab, plist, pcnt, zrows)


def _k2_body(deg_ref, x_ref, o_ref, ns_ref, nd_ref):
    d = deg_ref[...]
    ns = lax.rsqrt(jnp.maximum(d[:, 0:1], 1.0))
    nd = lax.rsqrt(jnp.maximum(d[:, 1:2], 1.0))
    o_ref[...] = x_ref[...] * ns
    ns_ref[...] = ns
    nd_ref[...] = nd


def _k2(deg_t, xp):
    return pl.pallas_call(
        _k2_body,
        grid=(GRID2,),
        in_specs=[
            pl.BlockSpec((RB, NC), lambda i: (i, 0)),
            pl.BlockSpec((RB, F1), lambda i: (i, 0)),
        ],
        out_specs=[
            pl.BlockSpec((RB, F1), lambda i: (i, 0)),
            pl.BlockSpec((RB, 1), lambda i: (i, 0)),
            pl.BlockSpec((RB, 1), lambda i: (i, 0)),
        ],
        out_shape=[
            jax.ShapeDtypeStruct((NAGG, F1), jnp.float32),
            jax.ShapeDtypeStruct((NAGG, 1), jnp.float32),
            jax.ShapeDtypeStruct((NAGG, 1), jnp.float32),
        ],
    )(deg_t, xp)


def _k4_body(a_ref, ns_ref, nd_ref, w1_ref, w2_ref, p_ref):
    z = jnp.dot(a_ref[...] * nd_ref[...], w1_ref[...],
                preferred_element_type=jnp.float32)
    z = jnp.maximum(z, 0.0) * ns_ref[...]
    p_ref[...] = jnp.dot(z, w2_ref[...], preferred_element_type=jnp.float32)


def _k4(agg1, ns, nd, w1p, W2):
    return pl.pallas_call(
        _k4_body,
        grid=(GRID2,),
        in_specs=[
            pl.BlockSpec((RB, F1), lambda i: (i, 0)),
            pl.BlockSpec((RB, 1), lambda i: (i, 0)),
            pl.BlockSpec((RB, 1), lambda i: (i, 0)),
            pl.BlockSpec((F1, HID), lambda i: (0, 0)),
            pl.BlockSpec((HID, OUT), lambda i: (0, 0)),
        ],
        out_specs=pl.BlockSpec((RB, OUT), lambda i: (i, 0)),
        out_shape=jax.ShapeDtypeStruct((NAGG, OUT), jnp.float32),
    )(agg1, ns, nd, w1p, W2)


def _k6_body(a_ref, nd_ref, o_ref):
    i = pl.program_id(0)
    m = jnp.max(a_ref[...] * nd_ref[...], axis=0, keepdims=True)

    @pl.when(i == 0)
    def _():
        o_ref[...] = m

    @pl.when(i > 0)
    def _():
        o_ref[...] = jnp.maximum(o_ref[...], m)

    @pl.when(i == GRID - 1)
    def _():
        o_ref[...] = jnp.maximum(o_ref[...], 0.0)


def _k6(agg2, nd):
    return pl.pallas_call(
        _k6_body,
        grid=(GRID,),
        in_specs=[
            pl.BlockSpec((RB, OUT), lambda i: (i, 0)),
            pl.BlockSpec((RB, 1), lambda i: (i, 0)),
        ],
        out_specs=pl.BlockSpec((1, OUT), lambda i: (0, 0)),
        out_shape=jax.ShapeDtypeStruct((1, OUT), jnp.float32),
    )(agg2, nd)


def kernel(x, edge_index, W1, W2):
    ei = edge_index.astype(jnp.int32)
    src = ei[0]
    dst = ei[1]

    npad = E_PAD - E
    ar = jnp.arange(npad, dtype=jnp.int32)
    trash = N + (ar % 1024)
    src_deg2d = jnp.concatenate([src, trash]).reshape(ROWS, EPR)
    src_edge2d = jnp.concatenate([src, ar % 1024]).reshape(ROWS, EPR)
    dst2d = jnp.concatenate([dst, trash]).reshape(ROWS, EPR)

    xp = jnp.pad(x, ((0, NAGG - N), (0, F1 - IN_F)))
    w1p = jnp.pad(W1, ((0, F1 - IN_F), (0, 0)))
    z96 = jnp.zeros((WZR, F1), jnp.float32)
    z64 = jnp.zeros((WZR, OUT), jnp.float32)

    deg = _deg_pass(src_deg2d, dst2d)
    deg_t = deg.T
    plist, pcnt = _partition(src_edge2d, dst2d)
    xn, ns, nd = _k2(deg_t, xp)
    agg1 = _edge_pass(xn, plist, pcnt, z96, F1).reshape(NAGG, F1)
    p = _k4(agg1, ns, nd, w1p, W2)
    agg2 = _edge_pass(p, plist, pcnt, z64, OUT).reshape(NAGG, OUT)
    return _k6(agg2, nd)


# speedup vs baseline: 7.5491x; 1.0345x over previous
import functools

import jax
import jax.numpy as jnp
from jax import lax
from jax.experimental import pallas as pl
from jax.experimental.pallas import tpu as pltpu
from jax.experimental.pallas import tpu_sc as plsc

N = 50000
E = 800000
IN_F = 69
F1 = 96
HID = 128
OUT = 64

NC = 2
NT = 16

EPR = 128
E_PAD = 819200
ROWS = E_PAD // EPR
RPT = ROWS // NT
IDXR = 80
KG = 8
N_OUTER = RPT // IDXR
N_INNER = IDXR // KG

NAGG = 51200
RNG = 4
RROWS = NAGG // RNG
RTRASH = 8
RROWS_T = RROWS + RTRASH

RINGSZ = 11264
CAP = 224 * 256
PSTG = 16
EB = 64
NBUF = 4

STRIPE = NAGG // NT
RSTRIPE = RROWS // NT
WZR = 160
NWZ = RSTRIPE // WZR

RB = 400
GRID2 = NAGG // RB
GRID = N // RB


def _deg_pass(srcd2d, dst2d):
    mesh = plsc.VectorSubcoreMesh(core_axis_name="c", subcore_axis_name="s")

    @functools.partial(
        pl.kernel,
        out_type=jax.ShapeDtypeStruct((NC, NAGG), jnp.float32),
        mesh=mesh,
        compiler_params=pltpu.CompilerParams(use_tc_tiling_on_sc=False),
        scratch_types=[
            pltpu.VMEM((IDXR, EPR), jnp.int32),
            pltpu.VMEM((EPR,), jnp.float32),
            pltpu.VMEM((STRIPE,), jnp.float32),
            pltpu.VMEM_SHARED((NAGG,), jnp.float32),
            pltpu.SemaphoreType.DMA,
        ],
    )
    def k(src_h, dst_h, out_h, idxv, ones_v, zflat, deg_sh, sem):
        c = lax.axis_index("c")
        s = lax.axis_index("s")

        zero16 = jnp.zeros((16,), jnp.float32)
        one16 = jnp.ones((16,), jnp.float32)

        def zfill(i, _):
            zflat[pl.ds(i * 16, 16)] = zero16
            return 0

        lax.fori_loop(0, STRIPE // 16, zfill, 0)
        for b in range(EPR // 16):
            ones_v[pl.ds(b * 16, 16)] = one16
        pltpu.sync_copy(zflat, deg_sh.at[pl.ds(s * STRIPE, STRIPE)])
        plsc.subcore_barrier()

        def process(idx_h):
            def outer(o, _):
                r0 = s * RPT + o * IDXR
                pltpu.sync_copy(idx_h.at[pl.ds(r0, IDXR)], idxv)

                def inner(g, _):
                    descs = [
                        pltpu.async_copy(
                            ones_v, deg_sh.at[idxv.at[g * KG + b]], sem, add=True
                        )
                        for b in range(KG)
                    ]
                    for d in descs:
                        d.wait()
                    return 0

                lax.fori_loop(0, N_INNER, inner, 0)
                return 0

            lax.fori_loop(0, N_OUTER, outer, 0)

        @pl.when(c == 0)
        def _():
            process(src_h)

        @pl.when(c == 1)
        def _():
            process(dst_h)

        plsc.subcore_barrier()
        pltpu.sync_copy(
            deg_sh.at[pl.ds(s * STRIPE, STRIPE)], out_h.at[c, pl.ds(s * STRIPE, STRIPE)]
        )

    return k(srcd2d, dst2d)


def _partition(src2d, dst2d):
    mesh = plsc.VectorSubcoreMesh(core_axis_name="c", subcore_axis_name="s")

    @functools.partial(
        pl.kernel,
        out_type=[
            jax.ShapeDtypeStruct((NC, NT, 2, CAP), jnp.int32),
            jax.ShapeDtypeStruct((NC, NT, 2, 16), jnp.int32),
        ],
        mesh=mesh,
        compiler_params=pltpu.CompilerParams(
            use_tc_tiling_on_sc=False, needs_layout_passes=False),
        scratch_types=[
            pltpu.VMEM((IDXR, EPR), jnp.int32),
            pltpu.VMEM((IDXR, EPR), jnp.int32),
            pltpu.VMEM((RINGSZ,), jnp.int32),
            pltpu.VMEM((RINGSZ,), jnp.int32),
            pltpu.VMEM((2, 16), jnp.int32),
        ],
    )
    def k(src_h, dst_h, plist_h, pcnt_h, sidx, didx, ring0, ring1, cntv):
        c = lax.axis_index("c")
        s = lax.axis_index("s")
        lo0 = (2 * c) * RROWS
        lo0s = lo0 * 65536
        los = RROWS * 65536
        tmask = jnp.ones((16,), jnp.bool_)
        trash16 = jnp.full((16,), RROWS * 65536, jnp.int32)

        rings = (ring0, ring1)

        def count(m):
            return jnp.max(plsc.all_reduce_population_count(m))

        def flush(ring, j, pos, fl):
            full = pos // 256

            def fk(kk, _):
                pltpu.sync_copy(
                    ring.at[pl.ds(kk * 256, 256)],
                    plist_h.at[c, s, j, pl.ds((fl + kk) * 256, 256)],
                )
                return 0

            lax.fori_loop(0, full, fk, 0)

            @pl.when(full > 0)
            def _():
                for t in range(16):
                    v = ring[pl.ds(full * 256 + 16 * t, 16)]
                    ring[pl.ds(16 * t, 16)] = v

            return pos - full * 256, fl + full

        def outer(o, carry):
            pos0, fl0, pos1, fl1 = carry
            r0 = s * RPT + o * IDXR
            pltpu.sync_copy(src_h.at[pl.ds(r0, IDXR)], sidx)
            pltpu.sync_copy(dst_h.at[pl.ds(r0, IDXR)], didx)

            def crow(r, carry2):
                pos0, pos1 = carry2
                for m in range(EPR // 16):
                    sv = sidx[r, pl.ds(m * 16, 16)]
                    dv = didx[r, pl.ds(m * 16, 16)]
                    dvs = dv * 65536
                    m0 = (dv >= lo0) & (dv < lo0 + RROWS)
                    p0 = (dvs - lo0s) | sv
                    plsc.store_compressed(ring0.at[pl.ds(pos0, 16)], p0, mask=m0)
                    pos0 = pos0 + count(m0)
                    m1 = (dv >= lo0 + RROWS) & (dv < lo0 + 2 * RROWS)
                    p1 = (dvs - lo0s - los) | sv
                    plsc.store_compressed(ring1.at[pl.ds(pos1, 16)], p1, mask=m1)
                    pos1 = pos1 + count(m1)
                return (pos0, pos1)

            pos0, pos1 = lax.fori_loop(0, IDXR, crow, (pos0, pos1))
            pos0, fl0 = flush(ring0, 0, pos0, fl0)
            pos1, fl1 = flush(ring1, 1, pos1, fl1)
            return (pos0, fl0, pos1, fl1)

        z = jnp.int32(0)
        pos0, fl0, pos1, fl1 = lax.fori_loop(0, N_OUTER, outer, (z, z, z, z))

        for j, (ring, pos, fl) in enumerate(((ring0, pos0, fl0), (ring1, pos1, fl1))):
            for t in range(16):
                plsc.store_compressed(ring.at[pl.ds(pos + 16 * t, 16)], trash16, mask=tmask)

            @pl.when(pos > 0)
            def _():
                pltpu.sync_copy(
                    ring.at[pl.ds(0, 256)],
                    plist_h.at[c, s, j, pl.ds(fl * 256, 256)],
                )

            n2 = fl + jnp.minimum(pos, 1)
            cntv[j, :] = jnp.full((16,), 1, jnp.int32) * n2

        pltpu.sync_copy(cntv, pcnt_h.at[c, s])

    return k(src2d, dst2d)


def _edge_pass(tab, plist, pcnt, zrows, F):
    mesh = plsc.VectorSubcoreMesh(core_axis_name="c", subcore_axis_name="s")

    @functools.partial(
        pl.kernel,
        out_type=jax.ShapeDtypeStruct((RNG, RROWS, F), jnp.float32),
        mesh=mesh,
        compiler_params=pltpu.CompilerParams(use_tc_tiling_on_sc=False),
        scratch_types=[
            pltpu.VMEM((PSTG * 256,), jnp.int32),
            pltpu.VMEM((NBUF, EB), jnp.int32),
            pltpu.VMEM((NBUF, EB), jnp.int32),
            pltpu.VMEM((NBUF, EB, F), jnp.float32),
            pltpu.VMEM((WZR, F), jnp.float32),
            pltpu.VMEM((2, 16), jnp.int32),
            pltpu.VMEM_SHARED((RROWS_T, F), jnp.float32),
        ] + [pltpu.SemaphoreType.DMA] * (2 * NBUF),
    )
    def k(tab_h, plist_h, pcnt_h, z_h, out_h,
          pbuf, sidxb, didxb, rowsb, zbuf, cntv, agg, *sems):
        gsems = sems[:NBUF]
        ssems = sems[NBUF:]
        c = lax.axis_index("c")
        s = lax.axis_index("s")

        pltpu.sync_copy(z_h, zbuf)
        pltpu.sync_copy(pcnt_h.at[c, s], cntv)

        def zero_stripe():
            def zloop(i, _):
                pltpu.sync_copy(zbuf, agg.at[pl.ds(s * RSTRIPE + i * WZR, WZR)])
                return 0

            lax.fori_loop(0, NWZ, zloop, 0)

            @pl.when(s == 0)
            def _():
                pltpu.sync_copy(zbuf.at[pl.ds(0, RTRASH)], agg.at[pl.ds(RROWS, RTRASH)])

        def writeout(q):
            def wloop(i, _):
                off = s * RSTRIPE + i * WZR
                pltpu.sync_copy(agg.at[pl.ds(off, WZR)], out_h.at[q, pl.ds(off, WZR)])
                return 0

            lax.fori_loop(0, NWZ, wloop, 0)

        def unpack(b, P):
            for mc in range(EB // 16):
                pk = pbuf[pl.ds(b * EB + mc * 16, 16)]
                sidxb[P, pl.ds(mc * 16, 16)] = pk & 0xFFFF
                didxb[P, pl.ds(mc * 16, 16)] = lax.shift_right_logical(pk, 16)

        def fire_g(b, P):
            return pltpu.async_copy(tab_h.at[sidxb.at[P]], rowsb.at[P], gsems[P])

        def fire_s(P):
            return pltpu.async_copy(
                rowsb.at[P], agg.at[didxb.at[P]], ssems[P], add=True
            )

        def process(r):
            n2 = cntv[r, pl.ds(0, 16)][0]
            nfull = n2 // PSTG

            def souter(t, _):
                base = t * PSTG
                pltpu.sync_copy(
                    plist_h.at[c, s, r, pl.ds(base * 256, PSTG * 256)], pbuf
                )
                nb = PSTG * 256 // EB
                gds = [None] * nb
                sds = [None] * nb
                for b in range(nb):
                    P = b % NBUF
                    if b >= NBUF:
                        sds[b - NBUF].wait()
                    unpack(b, P)
                    gds[b] = fire_g(b, P)
                    if b >= 2:
                        Q = (b - 2) % NBUF
                        gds[b - 2].wait()
                        sds[b - 2] = fire_s(Q)
                for b in (nb - 2, nb - 1):
                    Q = b % NBUF
                    gds[b].wait()
                    sds[b] = fire_s(Q)
                for b in range(nb - NBUF, nb):
                    sds[b].wait()
                return 0

            lax.fori_loop(0, nfull, souter, 0)

            tbase = nfull * PSTG
            mm = n2 - tbase
            pltpu.sync_copy(
                plist_h.at[c, s, r, pl.ds(tbase * 256, PSTG * 256)], pbuf
            )

            def pair(jp, _):
                gp = [None] * NBUF
                for q in range(NBUF):
                    unpack(NBUF * jp + q, q)
                    gp[q] = fire_g(NBUF * jp + q, q)
                sp = [None] * NBUF
                for q in range(NBUF):
                    gp[q].wait()
                    sp[q] = fire_s(q)
                for q in range(NBUF):
                    sp[q].wait()
                return 0

            lax.fori_loop(0, mm, pair, 0)

        zero_stripe()
        plsc.subcore_barrier()
        process(0)
        plsc.subcore_barrier()
        writeout(2 * c)
        zero_stripe()
        plsc.subcore_barrier()
        process(1)
        plsc.subcore_barrier()
        writeout(2 * c + 1)

    return k(tab, plist, pcnt, zrows)


def _k2_body(deg_ref, x_ref, o_ref, ns_ref, nd_ref):
    d = deg_ref[...]
    ns = lax.rsqrt(jnp.maximum(d[:, 0:1], 1.0))
    nd = lax.rsqrt(jnp.maximum(d[:, 1:2], 1.0))
    o_ref[...] = x_ref[...] * ns
    ns_ref[...] = ns
    nd_ref[...] = nd


def _k2(deg_t, xp):
    return pl.pallas_call(
        _k2_body,
        grid=(GRID2,),
        in_specs=[
            pl.BlockSpec((RB, NC), lambda i: (i, 0)),
            pl.BlockSpec((RB, F1), lambda i: (i, 0)),
        ],
        out_specs=[
            pl.BlockSpec((RB, F1), lambda i: (i, 0)),
            pl.BlockSpec((RB, 1), lambda i: (i, 0)),
            pl.BlockSpec((RB, 1), lambda i: (i, 0)),
        ],
        out_shape=[
            jax.ShapeDtypeStruct((NAGG, F1), jnp.float32),
            jax.ShapeDtypeStruct((NAGG, 1), jnp.float32),
            jax.ShapeDtypeStruct((NAGG, 1), jnp.float32),
        ],
    )(deg_t, xp)


def _k4_body(a_ref, ns_ref, nd_ref, w1_ref, w2_ref, p_ref):
    z = jnp.dot(a_ref[...] * nd_ref[...], w1_ref[...],
                preferred_element_type=jnp.float32)
    z = jnp.maximum(z, 0.0) * ns_ref[...]
    p_ref[...] = jnp.dot(z, w2_ref[...], preferred_element_type=jnp.float32)


def _k4(agg1, ns, nd, w1p, W2):
    return pl.pallas_call(
        _k4_body,
        grid=(GRID2,),
        in_specs=[
            pl.BlockSpec((RB, F1), lambda i: (i, 0)),
            pl.BlockSpec((RB, 1), lambda i: (i, 0)),
            pl.BlockSpec((RB, 1), lambda i: (i, 0)),
            pl.BlockSpec((F1, HID), lambda i: (0, 0)),
            pl.BlockSpec((HID, OUT), lambda i: (0, 0)),
        ],
        out_specs=pl.BlockSpec((RB, OUT), lambda i: (i, 0)),
        out_shape=jax.ShapeDtypeStruct((NAGG, OUT), jnp.float32),
    )(agg1, ns, nd, w1p, W2)


def _k6_body(a_ref, nd_ref, o_ref):
    i = pl.program_id(0)
    m = jnp.max(a_ref[...] * nd_ref[...], axis=0, keepdims=True)

    @pl.when(i == 0)
    def _():
        o_ref[...] = m

    @pl.when(i > 0)
    def _():
        o_ref[...] = jnp.maximum(o_ref[...], m)

    @pl.when(i == GRID - 1)
    def _():
        o_ref[...] = jnp.maximum(o_ref[...], 0.0)


def _k6(agg2, nd):
    return pl.pallas_call(
        _k6_body,
        grid=(GRID,),
        in_specs=[
            pl.BlockSpec((RB, OUT), lambda i: (i, 0)),
            pl.BlockSpec((RB, 1), lambda i: (i, 0)),
        ],
        out_specs=pl.BlockSpec((1, OUT), lambda i: (0, 0)),
        out_shape=jax.ShapeDtypeStruct((1, OUT), jnp.float32),
    )(agg2, nd)


def kernel(x, edge_index, W1, W2):
    ei = edge_index.astype(jnp.int32)
    src = ei[0]
    dst = ei[1]

    npad = E_PAD - E
    ar = jnp.arange(npad, dtype=jnp.int32)
    trash = N + (ar % 1024)
    src_deg2d = jnp.concatenate([src, trash]).reshape(ROWS, EPR)
    src_edge2d = jnp.concatenate([src, ar % 1024]).reshape(ROWS, EPR)
    dst2d = jnp.concatenate([dst, trash]).reshape(ROWS, EPR)

    xp = jnp.pad(x, ((0, NAGG - N), (0, F1 - IN_F)))
    w1p = jnp.pad(W1, ((0, F1 - IN_F), (0, 0)))
    z96 = jnp.zeros((WZR, F1), jnp.float32)
    z64 = jnp.zeros((WZR, OUT), jnp.float32)

    deg = _deg_pass(src_deg2d, dst2d)
    deg_t = deg.T
    plist, pcnt = _partition(src_edge2d, dst2d)
    xn, ns, nd = _k2(deg_t, xp)
    agg1 = _edge_pass(xn, plist, pcnt, z96, F1).reshape(NAGG, F1)
    p = _k4(agg1, ns, nd, w1p, W2)
    agg2 = _edge_pass(p, plist, pcnt, z64, OUT).reshape(NAGG, OUT)
    return _k6(agg2, nd)


# speedup vs baseline: 7.9042x; 1.0470x over previous
import functools

import jax
import jax.numpy as jnp
from jax import lax
from jax.experimental import pallas as pl
from jax.experimental.pallas import tpu as pltpu
from jax.experimental.pallas import tpu_sc as plsc

N = 50000
E = 800000
IN_F = 69
F1 = 96
HID = 128
OUT = 64

NC = 2
NT = 16

EPR = 128
E_PAD = 819200
ROWS = E_PAD // EPR
RPT = ROWS // NT
IDXR = 80
KG = 8
N_OUTER = RPT // IDXR
N_INNER = IDXR // KG

NAGG = 51200
RNG = 4
RROWS = NAGG // RNG
RTRASH = 8
RROWS_T = RROWS + RTRASH

RINGSZ = 11264
CAP = 224 * 256
PSTG = 16
EB = 64
NBUF = 4

STRIPE = NAGG // NT
RSTRIPE = RROWS // NT
WZR = 160
NWZ = RSTRIPE // WZR

RB = 512
GRID2 = NAGG // RB


def _deg_pass(srcd2d, dst2d):
    mesh = plsc.VectorSubcoreMesh(core_axis_name="c", subcore_axis_name="s")

    @functools.partial(
        pl.kernel,
        out_type=jax.ShapeDtypeStruct((NC, NAGG), jnp.float32),
        mesh=mesh,
        compiler_params=pltpu.CompilerParams(use_tc_tiling_on_sc=False),
        scratch_types=[
            pltpu.VMEM((IDXR, EPR), jnp.int32),
            pltpu.VMEM((EPR,), jnp.float32),
            pltpu.VMEM((STRIPE,), jnp.float32),
            pltpu.VMEM_SHARED((NAGG,), jnp.float32),
            pltpu.SemaphoreType.DMA,
        ],
    )
    def k(src_h, dst_h, out_h, idxv, ones_v, zflat, deg_sh, sem):
        c = lax.axis_index("c")
        s = lax.axis_index("s")

        zero16 = jnp.zeros((16,), jnp.float32)
        one16 = jnp.ones((16,), jnp.float32)

        def zfill(i, _):
            zflat[pl.ds(i * 16, 16)] = zero16
            return 0

        lax.fori_loop(0, STRIPE // 16, zfill, 0)
        for b in range(EPR // 16):
            ones_v[pl.ds(b * 16, 16)] = one16
        pltpu.sync_copy(zflat, deg_sh.at[pl.ds(s * STRIPE, STRIPE)])
        plsc.subcore_barrier()

        def process(idx_h):
            def outer(o, _):
                r0 = s * RPT + o * IDXR
                pltpu.sync_copy(idx_h.at[pl.ds(r0, IDXR)], idxv)

                def inner(g, _):
                    descs = [
                        pltpu.async_copy(
                            ones_v, deg_sh.at[idxv.at[g * KG + b]], sem, add=True
                        )
                        for b in range(KG)
                    ]
                    for d in descs:
                        d.wait()
                    return 0

                lax.fori_loop(0, N_INNER, inner, 0)
                return 0

            lax.fori_loop(0, N_OUTER, outer, 0)

        @pl.when(c == 0)
        def _():
            process(src_h)

        @pl.when(c == 1)
        def _():
            process(dst_h)

        plsc.subcore_barrier()
        pltpu.sync_copy(
            deg_sh.at[pl.ds(s * STRIPE, STRIPE)], out_h.at[c, pl.ds(s * STRIPE, STRIPE)]
        )

    return k(srcd2d, dst2d)


def _partition(src2d, dst2d):
    mesh = plsc.VectorSubcoreMesh(core_axis_name="c", subcore_axis_name="s")

    @functools.partial(
        pl.kernel,
        out_type=[
            jax.ShapeDtypeStruct((NC, NT, 2, CAP), jnp.int32),
            jax.ShapeDtypeStruct((NC, NT, 2, 16), jnp.int32),
        ],
        mesh=mesh,
        compiler_params=pltpu.CompilerParams(
            use_tc_tiling_on_sc=False, needs_layout_passes=False),
        scratch_types=[
            pltpu.VMEM((IDXR, EPR), jnp.int32),
            pltpu.VMEM((IDXR, EPR), jnp.int32),
            pltpu.VMEM((RINGSZ,), jnp.int32),
            pltpu.VMEM((RINGSZ,), jnp.int32),
            pltpu.VMEM((2, 16), jnp.int32),
        ],
    )
    def k(src_h, dst_h, plist_h, pcnt_h, sidx, didx, ring0, ring1, cntv):
        c = lax.axis_index("c")
        s = lax.axis_index("s")
        lo0 = (2 * c) * RROWS
        lo0s = lo0 * 65536
        los = RROWS * 65536
        tmask = jnp.ones((16,), jnp.bool_)
        trash16 = jnp.full((16,), RROWS * 65536, jnp.int32)

        rings = (ring0, ring1)

        def count(m):
            return jnp.max(plsc.all_reduce_population_count(m))

        def flush(ring, j, pos, fl):
            full = pos // 256

            def fk(kk, _):
                pltpu.sync_copy(
                    ring.at[pl.ds(kk * 256, 256)],
                    plist_h.at[c, s, j, pl.ds((fl + kk) * 256, 256)],
                )
                return 0

            lax.fori_loop(0, full, fk, 0)

            @pl.when(full > 0)
            def _():
                for t in range(16):
                    v = ring[pl.ds(full * 256 + 16 * t, 16)]
                    ring[pl.ds(16 * t, 16)] = v

            return pos - full * 256, fl + full

        def outer(o, carry):
            pos0, fl0, pos1, fl1 = carry
            r0 = s * RPT + o * IDXR
            pltpu.sync_copy(src_h.at[pl.ds(r0, IDXR)], sidx)
            pltpu.sync_copy(dst_h.at[pl.ds(r0, IDXR)], didx)

            def crow(r, carry2):
                pos0, pos1 = carry2
                for m in range(EPR // 16):
                    sv = sidx[r, pl.ds(m * 16, 16)]
                    dv = didx[r, pl.ds(m * 16, 16)]
                    dvs = dv * 65536
                    m0 = (dv >= lo0) & (dv < lo0 + RROWS)
                    p0 = (dvs - lo0s) | sv
                    plsc.store_compressed(ring0.at[pl.ds(pos0, 16)], p0, mask=m0)
                    pos0 = pos0 + count(m0)
                    m1 = (dv >= lo0 + RROWS) & (dv < lo0 + 2 * RROWS)
                    p1 = (dvs - lo0s - los) | sv
                    plsc.store_compressed(ring1.at[pl.ds(pos1, 16)], p1, mask=m1)
                    pos1 = pos1 + count(m1)
                return (pos0, pos1)

            pos0, pos1 = lax.fori_loop(0, IDXR, crow, (pos0, pos1))
            pos0, fl0 = flush(ring0, 0, pos0, fl0)
            pos1, fl1 = flush(ring1, 1, pos1, fl1)
            return (pos0, fl0, pos1, fl1)

        z = jnp.int32(0)
        pos0, fl0, pos1, fl1 = lax.fori_loop(0, N_OUTER, outer, (z, z, z, z))

        for j, (ring, pos, fl) in enumerate(((ring0, pos0, fl0), (ring1, pos1, fl1))):
            for t in range(16):
                plsc.store_compressed(ring.at[pl.ds(pos + 16 * t, 16)], trash16, mask=tmask)

            @pl.when(pos > 0)
            def _():
                pltpu.sync_copy(
                    ring.at[pl.ds(0, 256)],
                    plist_h.at[c, s, j, pl.ds(fl * 256, 256)],
                )

            n2 = fl + jnp.minimum(pos, 1)
            cntv[j, :] = jnp.full((16,), 1, jnp.int32) * n2

        pltpu.sync_copy(cntv, pcnt_h.at[c, s])

    return k(src2d, dst2d)


def _edge_pass(tab, plist, pcnt, zrows, F):
    mesh = plsc.VectorSubcoreMesh(core_axis_name="c", subcore_axis_name="s")

    @functools.partial(
        pl.kernel,
        out_type=jax.ShapeDtypeStruct((RNG, RROWS, F), jnp.float32),
        mesh=mesh,
        compiler_params=pltpu.CompilerParams(use_tc_tiling_on_sc=False),
        scratch_types=[
            pltpu.VMEM((PSTG * 256,), jnp.int32),
            pltpu.VMEM((NBUF, EB), jnp.int32),
            pltpu.VMEM((NBUF, EB), jnp.int32),
            pltpu.VMEM((NBUF, EB, F), jnp.float32),
            pltpu.VMEM((WZR, F), jnp.float32),
            pltpu.VMEM((2, 16), jnp.int32),
            pltpu.VMEM_SHARED((RROWS_T, F), jnp.float32),
        ] + [pltpu.SemaphoreType.DMA] * (2 * NBUF),
    )
    def k(tab_h, plist_h, pcnt_h, z_h, out_h,
          pbuf, sidxb, didxb, rowsb, zbuf, cntv, agg, *sems):
        gsems = sems[:NBUF]
        ssems = sems[NBUF:]
        c = lax.axis_index("c")
        s = lax.axis_index("s")

        pltpu.sync_copy(z_h, zbuf)
        pltpu.sync_copy(pcnt_h.at[c, s], cntv)

        def zero_stripe():
            def zloop(i, _):
                pltpu.sync_copy(zbuf, agg.at[pl.ds(s * RSTRIPE + i * WZR, WZR)])
                return 0

            lax.fori_loop(0, NWZ, zloop, 0)

            @pl.when(s == 0)
            def _():
                pltpu.sync_copy(zbuf.at[pl.ds(0, RTRASH)], agg.at[pl.ds(RROWS, RTRASH)])

        def writeout(q):
            def wloop(i, _):
                off = s * RSTRIPE + i * WZR
                pltpu.sync_copy(agg.at[pl.ds(off, WZR)], out_h.at[q, pl.ds(off, WZR)])
                return 0

            lax.fori_loop(0, NWZ, wloop, 0)

        def unpack(b, P):
            for mc in range(EB // 16):
                pk = pbuf[pl.ds(b * EB + mc * 16, 16)]
                sidxb[P, pl.ds(mc * 16, 16)] = pk & 0xFFFF
                didxb[P, pl.ds(mc * 16, 16)] = lax.shift_right_logical(pk, 16)

        def fire_g(b, P):
            return pltpu.async_copy(tab_h.at[sidxb.at[P]], rowsb.at[P], gsems[P])

        def fire_s(P):
            return pltpu.async_copy(
                rowsb.at[P], agg.at[didxb.at[P]], ssems[P], add=True
            )

        def process(r):
            n2 = cntv[r, pl.ds(0, 16)][0]
            nfull = n2 // PSTG

            def souter(t, _):
                base = t * PSTG
                pltpu.sync_copy(
                    plist_h.at[c, s, r, pl.ds(base * 256, PSTG * 256)], pbuf
                )
                nb = PSTG * 256 // EB
                gds = [None] * nb
                sds = [None] * nb
                for b in range(nb):
                    P = b % NBUF
                    if b >= NBUF:
                        sds[b - NBUF].wait()
                    unpack(b, P)
                    gds[b] = fire_g(b, P)
                    if b >= 2:
                        Q = (b - 2) % NBUF
                        gds[b - 2].wait()
                        sds[b - 2] = fire_s(Q)
                for b in (nb - 2, nb - 1):
                    Q = b % NBUF
                    gds[b].wait()
                    sds[b] = fire_s(Q)
                for b in range(nb - NBUF, nb):
                    sds[b].wait()
                return 0

            lax.fori_loop(0, nfull, souter, 0)

            tbase = nfull * PSTG
            mm = n2 - tbase
            pltpu.sync_copy(
                plist_h.at[c, s, r, pl.ds(tbase * 256, PSTG * 256)], pbuf
            )

            def pair(jp, _):
                gp = [None] * NBUF
                for q in range(NBUF):
                    unpack(NBUF * jp + q, q)
                    gp[q] = fire_g(NBUF * jp + q, q)
                sp = [None] * NBUF
                for q in range(NBUF):
                    gp[q].wait()
                    sp[q] = fire_s(q)
                for q in range(NBUF):
                    sp[q].wait()
                return 0

            lax.fori_loop(0, mm, pair, 0)

        zero_stripe()
        plsc.subcore_barrier()
        process(0)
        plsc.subcore_barrier()
        writeout(2 * c)
        zero_stripe()
        plsc.subcore_barrier()
        process(1)
        plsc.subcore_barrier()
        writeout(2 * c + 1)

    return k(tab, plist, pcnt, zrows)


def _k2_body(deg_ref, x_ref, o_ref, ns_ref, nd_ref):
    d = jnp.transpose(deg_ref[...])
    ns = lax.rsqrt(jnp.maximum(d[:, 0:1], 1.0))
    nd = lax.rsqrt(jnp.maximum(d[:, 1:2], 1.0))
    o_ref[...] = x_ref[...] * ns
    ns_ref[...] = ns
    nd_ref[...] = nd


def _k2(deg, xp):
    return pl.pallas_call(
        _k2_body,
        grid=(GRID2,),
        in_specs=[
            pl.BlockSpec((NC, RB), lambda i: (0, i)),
            pl.BlockSpec((RB, F1), lambda i: (i, 0)),
        ],
        out_specs=[
            pl.BlockSpec((RB, F1), lambda i: (i, 0)),
            pl.BlockSpec((RB, 1), lambda i: (i, 0)),
            pl.BlockSpec((RB, 1), lambda i: (i, 0)),
        ],
        out_shape=[
            jax.ShapeDtypeStruct((NAGG, F1), jnp.float32),
            jax.ShapeDtypeStruct((NAGG, 1), jnp.float32),
            jax.ShapeDtypeStruct((NAGG, 1), jnp.float32),
        ],
    )(deg, xp)


def _k4_body(a_ref, ns_ref, nd_ref, w1_ref, w2_ref, p_ref):
    z = jnp.dot(a_ref[...] * nd_ref[...], w1_ref[...],
                preferred_element_type=jnp.float32)
    z = jnp.maximum(z, 0.0) * ns_ref[...]
    p_ref[...] = jnp.dot(z, w2_ref[...], preferred_element_type=jnp.float32)


def _k4(agg1, ns, nd, w1p, W2):
    return pl.pallas_call(
        _k4_body,
        grid=(GRID2,),
        in_specs=[
            pl.BlockSpec((RB, F1), lambda i: (i, 0)),
            pl.BlockSpec((RB, 1), lambda i: (i, 0)),
            pl.BlockSpec((RB, 1), lambda i: (i, 0)),
            pl.BlockSpec((F1, HID), lambda i: (0, 0)),
            pl.BlockSpec((HID, OUT), lambda i: (0, 0)),
        ],
        out_specs=pl.BlockSpec((RB, OUT), lambda i: (i, 0)),
        out_shape=jax.ShapeDtypeStruct((NAGG, OUT), jnp.float32),
    )(agg1, ns, nd, w1p, W2)


def _k6_body(a_ref, nd_ref, o_ref):
    i = pl.program_id(0)
    z = a_ref[...] * nd_ref[...]
    gid = lax.broadcasted_iota(jnp.int32, (RB, 1), 0) + i * RB
    z = jnp.where(gid < N, z, -3.0e38)
    m = jnp.max(z, axis=0, keepdims=True)

    @pl.when(i == 0)
    def _():
        o_ref[...] = m

    @pl.when(i > 0)
    def _():
        o_ref[...] = jnp.maximum(o_ref[...], m)

    @pl.when(i == GRID2 - 1)
    def _():
        o_ref[...] = jnp.maximum(o_ref[...], 0.0)


def _k6(agg2, nd):
    return pl.pallas_call(
        _k6_body,
        grid=(GRID2,),
        in_specs=[
            pl.BlockSpec((RB, OUT), lambda i: (i, 0)),
            pl.BlockSpec((RB, 1), lambda i: (i, 0)),
        ],
        out_specs=pl.BlockSpec((1, OUT), lambda i: (0, 0)),
        out_shape=jax.ShapeDtypeStruct((1, OUT), jnp.float32),
    )(agg2, nd)


def kernel(x, edge_index, W1, W2):
    ei = edge_index.astype(jnp.int32)
    src = ei[0]
    dst = ei[1]

    npad = E_PAD - E
    ar = jnp.arange(npad, dtype=jnp.int32)
    trash = N + (ar % 1024)
    src_deg2d = jnp.concatenate([src, trash]).reshape(ROWS, EPR)
    src_edge2d = jnp.concatenate([src, ar % 1024]).reshape(ROWS, EPR)
    dst2d = jnp.concatenate([dst, trash]).reshape(ROWS, EPR)

    xp = jnp.pad(x, ((0, NAGG - N), (0, F1 - IN_F)))
    w1p = jnp.pad(W1, ((0, F1 - IN_F), (0, 0)))
    z96 = jnp.zeros((WZR, F1), jnp.float32)
    z64 = jnp.zeros((WZR, OUT), jnp.float32)

    deg = _deg_pass(src_deg2d, dst2d)
    plist, pcnt = _partition(src_edge2d, dst2d)
    xn, ns, nd = _k2(deg, xp)
    agg1 = _edge_pass(xn, plist, pcnt, z96, F1).reshape(NAGG, F1)
    p = _k4(agg1, ns, nd, w1p, W2)
    agg2 = _edge_pass(p, plist, pcnt, z64, OUT).reshape(NAGG, OUT)
    return _k6(agg2, nd)
